# baseline (speedup 1.0000x reference)
import functools

import jax
import jax.numpy as jnp
from jax.experimental import pallas as pl
from jax.experimental.pallas import tpu as pltpu

_BS, _L, _D = 128, 12, 8192


def _body(idx_ref, drop_ref, in_ref, out_ref):
    b = pl.program_id(0)
    idx = idx_ref[b]
    dropped = drop_ref[b] > (1.0 - 0.2)
    l = jax.lax.broadcasted_iota(jnp.int32, (_L, 1), 0)
    cond = jnp.logical_and(dropped, l == idx)
    out_ref[0] = jnp.where(cond, jnp.float32(0.0), in_ref[0])


@jax.jit
def kernel(inputs, index, drop_rand):
    grid_spec = pltpu.PrefetchScalarGridSpec(
        num_scalar_prefetch=2,
        grid=(_BS,),
        in_specs=[
            pl.BlockSpec((1, _L, _D), lambda b, idx_ref, drop_ref: (b, 0, 0)),
        ],
        out_specs=pl.BlockSpec((1, _L, _D), lambda b, idx_ref, drop_ref: (b, 0, 0)),
    )
    return pl.pallas_call(
        _body,
        grid_spec=grid_spec,
        out_shape=jax.ShapeDtypeStruct((_BS, _L, _D), jnp.float32),
        compiler_params=pltpu.CompilerParams(
            dimension_semantics=("arbitrary",),
        ),
    )(index, drop_rand, inputs)


# speedup vs baseline: 1.3857x; 1.3857x over previous
import functools

import jax
import jax.numpy as jnp
from jax.experimental import pallas as pl
from jax.experimental.pallas import tpu as pltpu

_BS, _L, _D = 128, 12, 8192


_BB = 8


def _body(idx_ref, drop_ref, in_ref, out_ref):
    p = pl.program_id(0)
    l = jax.lax.broadcasted_iota(jnp.int32, (_L, 1), 0)
    for k in range(_BB):
        b = p * _BB + k
        idx = idx_ref[b]
        dropped = drop_ref[b] > (1.0 - 0.2)
        cond = jnp.logical_and(dropped, l == idx)
        out_ref[k] = jnp.where(cond, jnp.float32(0.0), in_ref[k])


@jax.jit
def kernel(inputs, index, drop_rand):
    grid_spec = pltpu.PrefetchScalarGridSpec(
        num_scalar_prefetch=2,
        grid=(_BS // _BB,),
        in_specs=[
            pl.BlockSpec((_BB, _L, _D), lambda b, idx_ref, drop_ref: (b, 0, 0)),
        ],
        out_specs=pl.BlockSpec((_BB, _L, _D), lambda b, idx_ref, drop_ref: (b, 0, 0)),
    )
    return pl.pallas_call(
        _body,
        grid_spec=grid_spec,
        out_shape=jax.ShapeDtypeStruct((_BS, _L, _D), jnp.float32),
        compiler_params=pltpu.CompilerParams(
            dimension_semantics=("arbitrary",),
        ),
    )(index, drop_rand, inputs)
